# baseline (speedup 1.0000x reference)
import functools

import jax
import jax.numpy as jnp
from jax import lax
from jax.experimental import pallas as pl
from jax.experimental.pallas import tpu as pltpu
from jax.experimental.pallas import tpu_sc as plsc

NC = 2
NS = 16
NW = NC * NS

NI = 4096
NJ = 200
C = 32
IB = NI // NW
L = 16

_MESH = plsc.VectorSubcoreMesh(
    core_axis_name="c", subcore_axis_name="s", num_cores=NC, num_subcores=NS
)


@functools.partial(
    pl.kernel,
    out_type=jax.ShapeDtypeStruct((NJ, C, NI), jnp.float32),
    mesh=_MESH,
    scratch_types=[
        [pltpu.VMEM((IB,), jnp.int32) for _ in range(2)],
        [pltpu.VMEM((IB,), jnp.int32) for _ in range(2)],
        [pltpu.VMEM((IB,), jnp.int32) for _ in range(2)],
        [pltpu.VMEM((IB, 128), jnp.float32) for _ in range(2)],
        pltpu.VMEM((C, IB), jnp.float32),
        [pltpu.SemaphoreType.DMA for _ in range(2)],
    ],
    compiler_params=pltpu.CompilerParams(use_tc_tiling_on_sc=True, needs_layout_passes=False),
)
def _grid_gather(tt_hbm, table_hbm, out_hbm, idx_v, rows_id, sub32, lines, slab, sems):
    wid = lax.axis_index("s") * NC + lax.axis_index("c")
    i0 = wid * IB
    iota = lax.iota(jnp.int32, L)

    def stage(j, b):
        pltpu.sync_copy(tt_hbm.at[pl.ds(j * NI + i0, IB)], idx_v[b])
        for k in range(IB // L):
            v = idx_v[b][pl.ds(k * L, L)]
            rows_id[b][pl.ds(k * L, L)] = lax.shift_right_logical(v, 2)
            sub32[b][pl.ds(k * L, L)] = lax.shift_left(
                lax.bitwise_and(v, 3), 5
            )
        pltpu.async_copy(table_hbm.at[rows_id[b]], lines[b], sems[b])

    stage(0, 0)

    def outer(g, carry):
        for b in range(2):
            j = 2 * g + b
            pltpu.make_async_copy(
                table_hbm.at[rows_id[b]], lines[b], sems[b]
            ).wait()

            @pl.when(j + 1 < NJ)
            def _():
                stage(j + 1, 1 - b)

            for k in range(IB // L):
                rowvec = k * L + iota
                colbase = sub32[b][pl.ds(k * L, L)]
                for c in range(C):
                    slab[c, pl.ds(k * L, L)] = plsc.load_gather(
                        lines[b], [rowvec, colbase + c]
                    )
            pltpu.sync_copy(slab, out_hbm.at[j, :, pl.ds(i0, IB)])
        return carry

    lax.fori_loop(0, NJ // 2, outer, 0)


def kernel(t, grid):
    tt = t.T.reshape(-1).astype(jnp.int32)
    table128 = grid.reshape(250000, 128)
    out3 = _grid_gather(tt, table128)
    return out3.transpose(2, 0, 1)


# speedup vs baseline: 1.1104x; 1.1104x over previous
import functools

import jax
import jax.numpy as jnp
from jax import lax
from jax.experimental import pallas as pl
from jax.experimental.pallas import tpu as pltpu
from jax.experimental.pallas import tpu_sc as plsc

NC = 2
NS = 16
NW = NC * NS

NI = 4096
NJ = 200
C = 32
IB = NI // NW
L = 16

_MESH = plsc.VectorSubcoreMesh(
    core_axis_name="c", subcore_axis_name="s", num_cores=NC, num_subcores=NS
)


@functools.partial(
    pl.kernel,
    out_type=jax.ShapeDtypeStruct((NJ, C, NI), jnp.float32),
    mesh=_MESH,
    scratch_types=[
        pltpu.VMEM((NJ, IB), jnp.int32),
        [pltpu.VMEM((IB,), jnp.int32) for _ in range(2)],
        [pltpu.VMEM((IB,), jnp.int32) for _ in range(2)],
        [pltpu.VMEM((IB, 128), jnp.float32) for _ in range(2)],
        [pltpu.VMEM((C, IB), jnp.float32) for _ in range(2)],
        pltpu.SemaphoreType.DMA,
        [pltpu.SemaphoreType.DMA for _ in range(2)],
        [pltpu.SemaphoreType.DMA for _ in range(2)],
    ],
    compiler_params=pltpu.CompilerParams(
        use_tc_tiling_on_sc=True, needs_layout_passes=False
    ),
)
def _grid_gather(
    tt_hbm, table_hbm, out_hbm,
    idx_v, rows_id, sub32, lines, slab,
    sem_idx, sem_g, sem_s,
):
    wid = lax.axis_index("s") * NC + lax.axis_index("c")
    i0 = wid * IB
    iota = lax.iota(jnp.int32, L)

    pltpu.async_copy(tt_hbm.at[:, pl.ds(i0, IB)], idx_v, sem_idx).wait()

    def fire(j, b):
        for k in range(IB // L):
            v = idx_v[j, pl.ds(k * L, L)]
            rows_id[b][pl.ds(k * L, L)] = lax.shift_right_logical(v, 2)
            sub32[b][pl.ds(k * L, L)] = lax.shift_left(lax.bitwise_and(v, 3), 5)
        pltpu.async_copy(table_hbm.at[rows_id[b]], lines[b], sem_g[b])

    fire(0, 0)
    fire(1, 1)

    def outer(g, carry):
        for b in range(2):
            j = 2 * g + b
            pltpu.make_async_copy(table_hbm.at[rows_id[b]], lines[b], sem_g[b]).wait()

            @pl.when(j >= 2)
            def _():
                pltpu.make_async_copy(
                    slab[b], out_hbm.at[j, :, pl.ds(i0, IB)], sem_s[b]
                ).wait()

            for k in range(IB // L):
                rowvec = k * L + iota
                colbase = sub32[b][pl.ds(k * L, L)]
                for c in range(C):
                    slab[b][c, pl.ds(k * L, L)] = plsc.load_gather(
                        lines[b], [rowvec, colbase + c]
                    )

            @pl.when(j + 2 < NJ)
            def _():
                fire(j + 2, b)

            pltpu.async_copy(slab[b], out_hbm.at[j, :, pl.ds(i0, IB)], sem_s[b])
        return carry

    lax.fori_loop(0, NJ // 2, outer, 0)

    for b in range(2):
        pltpu.make_async_copy(
            slab[b], out_hbm.at[NJ - 2 + b, :, pl.ds(i0, IB)], sem_s[b]
        ).wait()


def kernel(t, grid):
    tt2 = t.T.astype(jnp.int32)
    table128 = grid.reshape(250000, 128)
    out3 = _grid_gather(tt2, table128)
    return out3.transpose(2, 0, 1)


# speedup vs baseline: 1.3216x; 1.1903x over previous
import functools

import jax
import jax.numpy as jnp
from jax import lax
from jax.experimental import pallas as pl
from jax.experimental.pallas import tpu as pltpu
from jax.experimental.pallas import tpu_sc as plsc

NC = 2
NS = 16
NW = NC * NS

B = 4096 * 200
C = 32
N_PER_W = B // NW
CHUNK = 1600
NBUF = 2
N_CHUNKS = N_PER_W // CHUNK
N_OUTER = N_CHUNKS // NBUF

_MESH = plsc.VectorSubcoreMesh(
    core_axis_name="c", subcore_axis_name="s", num_cores=NC, num_subcores=NS
)


@functools.partial(
    pl.kernel,
    out_type=jax.ShapeDtypeStruct((B, C), jnp.float32),
    mesh=_MESH,
    scratch_types=[
        pltpu.VMEM((N_PER_W,), jnp.int32),
        [pltpu.VMEM((CHUNK, C), jnp.float32) for _ in range(NBUF)],
        [pltpu.SemaphoreType.DMA for _ in range(NBUF)],
    ],
    compiler_params=pltpu.CompilerParams(use_tc_tiling_on_sc=False),
)
def _grid_gather(idx_hbm, table_hbm, out_hbm, idx_v, rows, sems):
    wid = lax.axis_index("s") * NC + lax.axis_index("c")
    base = wid * N_PER_W

    pltpu.sync_copy(idx_hbm.at[pl.ds(base, N_PER_W)], idx_v)

    def fire(chunk, b):
        pltpu.async_copy(
            table_hbm.at[idx_v.at[pl.ds(chunk * CHUNK, CHUNK)]], rows[b], sems[b]
        )

    for b in range(NBUF):
        fire(b, b)

    def outer(g, carry):
        first = g * NBUF
        for b in range(NBUF):
            pltpu.make_async_copy(
                table_hbm.at[idx_v.at[pl.ds(0, CHUNK)]], rows[b], sems[b]
            ).wait()
            pltpu.sync_copy(rows[b], out_hbm.at[pl.ds(base + (first + b) * CHUNK, CHUNK)])
            nxt = first + b + NBUF

            @pl.when(nxt < N_CHUNKS)
            def _():
                fire(nxt, b)

        return carry

    lax.fori_loop(0, N_OUTER, outer, 0)


def kernel(t, grid):
    flat_idx = t.reshape(-1).astype(jnp.int32)
    out = _grid_gather(flat_idx, grid)
    return out.reshape(t.shape + (grid.shape[1],))


# speedup vs baseline: 1.3227x; 1.0008x over previous
import functools

import jax
import jax.numpy as jnp
from jax import lax
from jax.experimental import pallas as pl
from jax.experimental.pallas import tpu as pltpu
from jax.experimental.pallas import tpu_sc as plsc

NC = 2
NS = 16
NW = NC * NS

B = 4096 * 200
C = 32
N_PER_W = B // NW
CHUNK = 800
NBUF = 4
N_CHUNKS = N_PER_W // CHUNK
N_OUTER = N_CHUNKS // NBUF

_MESH = plsc.VectorSubcoreMesh(
    core_axis_name="c", subcore_axis_name="s", num_cores=NC, num_subcores=NS
)


@functools.partial(
    pl.kernel,
    out_type=jax.ShapeDtypeStruct((B, C), jnp.float32),
    mesh=_MESH,
    scratch_types=[
        pltpu.VMEM((N_PER_W,), jnp.int32),
        [pltpu.VMEM((CHUNK, C), jnp.float32) for _ in range(NBUF)],
        [pltpu.SemaphoreType.DMA for _ in range(NBUF)],
    ],
    compiler_params=pltpu.CompilerParams(use_tc_tiling_on_sc=False),
)
def _grid_gather(idx_hbm, table_hbm, out_hbm, idx_v, rows, sems):
    wid = lax.axis_index("s") * NC + lax.axis_index("c")
    base = wid * N_PER_W

    pltpu.sync_copy(idx_hbm.at[pl.ds(base, N_PER_W)], idx_v)

    def fire(chunk, b):
        pltpu.async_copy(
            table_hbm.at[idx_v.at[pl.ds(chunk * CHUNK, CHUNK)]], rows[b], sems[b]
        )

    for b in range(NBUF):
        fire(b, b)

    def outer(g, carry):
        first = g * NBUF
        for b in range(NBUF):
            pltpu.make_async_copy(
                table_hbm.at[idx_v.at[pl.ds(0, CHUNK)]], rows[b], sems[b]
            ).wait()
            pltpu.sync_copy(rows[b], out_hbm.at[pl.ds(base + (first + b) * CHUNK, CHUNK)])
            nxt = first + b + NBUF

            @pl.when(nxt < N_CHUNKS)
            def _():
                fire(nxt, b)

        return carry

    lax.fori_loop(0, N_OUTER, outer, 0)


def kernel(t, grid):
    flat_idx = t.reshape(-1).astype(jnp.int32)
    out = _grid_gather(flat_idx, grid)
    return out.reshape(t.shape + (grid.shape[1],))
